# baseline (speedup 1.0000x reference)
import jax
import jax.numpy as jnp
from jax.experimental import pallas as pl
from jax.experimental.pallas import tpu as pltpu

D_MODEL = 768
N_VOICES = 4


def _pe_add_kernel(ts_ref, vw_ref, x_ref, o_ref):
    ts = ts_ref[...]
    vw = vw_ref[...]
    pe = jnp.concatenate([ts, ts, ts, ts], axis=1) + vw
    o_ref[...] = x_ref[...] + pe[None]


def kernel(x, timestep_emb, voice_emb):
    B, L, D = x.shape
    T = L // N_VOICES
    W = N_VOICES * D
    xw = x.reshape(B, T, W)
    vw = voice_emb.reshape(1, W)
    ts = timestep_emb[:T]

    BT = 256
    grid = (T // BT, B)
    out = pl.pallas_call(
        _pe_add_kernel,
        grid=grid,
        in_specs=[
            pl.BlockSpec((BT, D), lambda i, b: (i, 0)),
            pl.BlockSpec((1, W), lambda i, b: (0, 0)),
            pl.BlockSpec((1, BT, W), lambda i, b: (b, i, 0)),
        ],
        out_specs=pl.BlockSpec((1, BT, W), lambda i, b: (b, i, 0)),
        out_shape=jax.ShapeDtypeStruct((B, T, W), x.dtype),
        compiler_params=pltpu.CompilerParams(
            dimension_semantics=("parallel", "parallel"),
        ),
    )(ts, vw, xw)
    return out.reshape(B, L, D)


# speedup vs baseline: 3.5607x; 3.5607x over previous
import jax
import jax.numpy as jnp
from jax.experimental import pallas as pl
from jax.experimental.pallas import tpu as pltpu

D_MODEL = 768
N_VOICES = 4


def _pe_add_kernel(ts_ref, v_ref, x_ref, o_ref, pe_ref):
    bt = ts_ref.shape[0]

    @pl.when(pl.program_id(1) == 0)
    def _build_pe():
        ts = ts_ref[...]
        t_pe = jnp.repeat(ts, N_VOICES, axis=0)
        v_pe = pltpu.repeat(v_ref[...], bt, axis=0)
        pe_ref[...] = t_pe + v_pe

    o_ref[...] = x_ref[...] + pe_ref[...][None]


def kernel(x, timestep_emb, voice_emb):
    B, L, D = x.shape
    T = L // N_VOICES
    ts = timestep_emb[:T]

    BT = 256
    BL = BT * N_VOICES
    grid = (T // BT, B)
    return pl.pallas_call(
        _pe_add_kernel,
        grid=grid,
        in_specs=[
            pl.BlockSpec((BT, D), lambda i, b: (i, 0)),
            pl.BlockSpec((N_VOICES, D), lambda i, b: (0, 0)),
            pl.BlockSpec((1, BL, D), lambda i, b: (b, i, 0)),
        ],
        out_specs=pl.BlockSpec((1, BL, D), lambda i, b: (b, i, 0)),
        out_shape=jax.ShapeDtypeStruct((B, L, D), x.dtype),
        scratch_shapes=[pltpu.VMEM((BL, D), jnp.float32)],
    )(ts, voice_emb, x)


# speedup vs baseline: 3.7802x; 1.0617x over previous
import jax
import jax.numpy as jnp
from jax.experimental import pallas as pl
from jax.experimental.pallas import tpu as pltpu

D_MODEL = 768
N_VOICES = 4


def _pe_add_kernel(ts_ref, v_ref, x_ref, o_ref, pe_ref):
    bt = ts_ref.shape[0]

    @pl.when(pl.program_id(1) == 0)
    def _build_pe():
        ts = ts_ref[...]
        t_pe = jnp.repeat(ts, N_VOICES, axis=0)
        v_pe = pltpu.repeat(v_ref[...], bt, axis=0)
        pe_ref[...] = t_pe + v_pe

    o_ref[...] = x_ref[...] + pe_ref[...][None]


def kernel(x, timestep_emb, voice_emb):
    B, L, D = x.shape
    T = L // N_VOICES
    ts = timestep_emb[:T]

    BT = 512
    BL = BT * N_VOICES
    grid = (T // BT, B)
    return pl.pallas_call(
        _pe_add_kernel,
        grid=grid,
        in_specs=[
            pl.BlockSpec((BT, D), lambda i, b: (i, 0)),
            pl.BlockSpec((N_VOICES, D), lambda i, b: (0, 0)),
            pl.BlockSpec((1, BL, D), lambda i, b: (b, i, 0)),
        ],
        out_specs=pl.BlockSpec((1, BL, D), lambda i, b: (b, i, 0)),
        out_shape=jax.ShapeDtypeStruct((B, L, D), x.dtype),
        scratch_shapes=[pltpu.VMEM((BL, D), jnp.float32)],
    )(ts, voice_emb, x)


# speedup vs baseline: 4.0334x; 1.0670x over previous
import jax
import jax.numpy as jnp
from jax.experimental import pallas as pl
from jax.experimental.pallas import tpu as pltpu

D_MODEL = 768
N_VOICES = 4


def _pe_add_kernel(ts_ref, v_ref, x_ref, o_ref, pe_ref):
    bt = ts_ref.shape[0]

    @pl.when(pl.program_id(1) == 0)
    def _build_pe():
        ts = ts_ref[...]
        t_pe = jnp.repeat(ts, N_VOICES, axis=0)
        v_pe = pltpu.repeat(v_ref[...], bt, axis=0)
        pe_ref[...] = t_pe + v_pe

    o_ref[...] = x_ref[...] + pe_ref[...][None]


def kernel(x, timestep_emb, voice_emb):
    B, L, D = x.shape
    T = L // N_VOICES
    ts = timestep_emb[:T]

    BT = 512
    BB = 2
    BL = BT * N_VOICES
    grid = (T // BT, B // BB)
    return pl.pallas_call(
        _pe_add_kernel,
        grid=grid,
        in_specs=[
            pl.BlockSpec((BT, D), lambda i, b: (i, 0)),
            pl.BlockSpec((N_VOICES, D), lambda i, b: (0, 0)),
            pl.BlockSpec((BB, BL, D), lambda i, b: (b, i, 0)),
        ],
        out_specs=pl.BlockSpec((BB, BL, D), lambda i, b: (b, i, 0)),
        out_shape=jax.ShapeDtypeStruct((B, L, D), x.dtype),
        scratch_shapes=[pltpu.VMEM((BL, D), jnp.float32)],
        compiler_params=pltpu.CompilerParams(
            vmem_limit_bytes=100 * 1024 * 1024,
        ),
    )(ts, voice_emb, x)
